# baseline (speedup 1.0000x reference)
import functools

import jax
import jax.numpy as jnp
from jax import lax
from jax.experimental import pallas as pl
from jax.experimental.pallas import tpu as pltpu
from jax.experimental.pallas import tpu_sc as plsc

PRIOR = 0.3
NUM_BINS = 64
BIN_WIDTH = 1.0 / NUM_BINS
FRAC_PRIOR = 1.0 / (2.0 * PRIOR)

NC = 2
NS = 16
L = 16
NW = NC * NS
HB = 128
BINS = 2 * HB


NPIECE = 4


def _sc_hist_body(logits_hbm, labels_hbm, out_hbm, x_v, lab_v, h2_v, h1_v,
                  *sems):
    n = logits_hbm.shape[0]
    chunk = n // NW
    piece = chunk // NPIECE
    wid = lax.axis_index("s") * NC + lax.axis_index("c")
    base = wid * chunk
    cps = []
    for p in range(NPIECE):
        lo = p * piece
        cx = pltpu.make_async_copy(logits_hbm.at[pl.ds(base + lo, piece)],
                                   x_v.at[pl.ds(lo, piece)], sems[2 * p])
        cl = pltpu.make_async_copy(labels_hbm.at[pl.ds(base + lo, piece)],
                                   lab_v.at[pl.ds(lo, piece)], sems[2 * p + 1])
        cx.start()
        cl.start()
        cps.append((cx, cl))

    zeros = jnp.zeros((L,), jnp.float32)

    @functools.partial(plsc.parallel_loop, 0, (L * BINS) // L, unroll=8)
    def _(j):
        h2_v[pl.ds(j * L, L)] = zeros

    lane_base = lax.iota(jnp.int32, L) * BINS
    one = jnp.full((L,), 1.0, jnp.float32)

    for p in range(NPIECE):
        cps[p][0].wait()
        cps[p][1].wait()
        lo_vec = p * (piece // L)

        @functools.partial(plsc.parallel_loop, lo_vec, lo_vec + piece // L,
                           unroll=8)
        def _(i):
            x = x_v[pl.ds(i * L, L)]
            lab = lab_v[pl.ds(i * L, L)]
            s = one / (one + jnp.exp(-x))
            t = s * 64.0
            idx = t.astype(jnp.int32)
            frac = t - idx.astype(jnp.float32)
            flat = lane_base + idx + lab * HB
            plsc.addupdate_scatter(h2_v, [flat], one - frac)
            plsc.addupdate_scatter(h2_v, [flat + 1], frac)

    for c in range(BINS // L):
        acc = h2_v[pl.ds(c * L, L)]
        for lane in range(1, L):
            acc = acc + h2_v[pl.ds(lane * BINS + c * L, L)]
        h1_v[pl.ds(c * L, L)] = acc

    pltpu.sync_copy(h1_v, out_hbm.at[wid])


def _sc_partial_hist(logits, labels):
    n = logits.shape[0]
    mesh = plsc.VectorSubcoreMesh(core_axis_name="c", subcore_axis_name="s")
    chunk = n // NW
    f = pl.kernel(
        _sc_hist_body,
        out_type=jax.ShapeDtypeStruct((NW, BINS), jnp.float32),
        mesh=mesh,
        scratch_types=[
            pltpu.VMEM((chunk,), jnp.float32),
            pltpu.VMEM((chunk,), jnp.int32),
            pltpu.VMEM((L * BINS,), jnp.float32),
            pltpu.VMEM((BINS,), jnp.float32),
        ] + [pltpu.SemaphoreType.DMA] * (2 * NPIECE),
        compiler_params=pltpu.CompilerParams(needs_layout_passes=False),
    )
    return f(logits, labels)


def _tc_loss_body(p_ref, o_ref):
    h = jnp.sum(p_ref[...], axis=0, keepdims=True) * BIN_WIDTH
    col = lax.broadcasted_iota(jnp.int32, (1, BINS), 1)
    valid_u = col < (NUM_BINS + 1)
    valid_p = (col >= HB) & (col < HB + NUM_BINS + 1)
    hu_sum = jnp.sum(jnp.where(valid_u, h, 0.0))
    hp_sum = jnp.sum(jnp.where(valid_p, h, 0.0))
    proxy_u = jnp.where(col == 0, 1.0 - PRIOR, 0.0) + jnp.where(
        col == NUM_BINS, PRIOR, 0.0)
    proxy_p = jnp.where(col == HB + NUM_BINS, 1.0, 0.0)
    lu = jnp.sum(
        jnp.where(valid_u, jnp.abs(h / (hu_sum + 1e-8) - proxy_u), 0.0))
    lp = jnp.sum(
        jnp.where(valid_p, jnp.abs(h / (hp_sum + 1e-8) - proxy_p), 0.0))
    o_ref[0, 0] = (lp + FRAC_PRIOR * lu) / (NUM_BINS + 1.0)


def _tc_loss(partials):
    f = pl.pallas_call(
        _tc_loss_body,
        out_shape=jax.ShapeDtypeStruct((1, 1), jnp.float32),
        in_specs=[pl.BlockSpec(memory_space=pltpu.VMEM)],
        out_specs=pl.BlockSpec(memory_space=pltpu.SMEM),
    )
    return f(partials)


@jax.jit
def kernel(logits, labels):
    labels_i32 = labels.astype(jnp.int32)
    partials = _sc_partial_hist(logits, labels_i32)
    out = _tc_loss(partials)
    return out[0, 0]


# speedup vs baseline: 1.0069x; 1.0069x over previous
import functools

import jax
import jax.numpy as jnp
from jax import lax
from jax.experimental import pallas as pl
from jax.experimental.pallas import tpu as pltpu
from jax.experimental.pallas import tpu_sc as plsc

PRIOR = 0.3
NUM_BINS = 64
BIN_WIDTH = 1.0 / NUM_BINS
FRAC_PRIOR = 1.0 / (2.0 * PRIOR)

NC = 2
NS = 16
L = 16
NW = NC * NS
HB = 128
BINS = 2 * HB


NPIECE = 4


def _sc_hist_body(logits_hbm, labels_hbm, out_hbm, x_v, lab_v, h2_v, h1_v,
                  *sems):
    n = logits_hbm.shape[0]
    chunk = n // NW
    piece = chunk // NPIECE
    wid = lax.axis_index("s") * NC + lax.axis_index("c")
    base = wid * chunk
    cp_x = pltpu.make_async_copy(logits_hbm.at[pl.ds(base, chunk)], x_v,
                                 sems[0])
    cp_l = pltpu.make_async_copy(labels_hbm.at[pl.ds(base, chunk)], lab_v,
                                 sems[1])
    cp_x.start()
    cp_l.start()

    zeros = jnp.zeros((L,), jnp.float32)

    @functools.partial(plsc.parallel_loop, 0, (L * BINS) // L, unroll=8)
    def _(j):
        h2_v[pl.ds(j * L, L)] = zeros

    lane_base = lax.iota(jnp.int32, L) * BINS
    one = jnp.full((L,), 1.0, jnp.float32)

    cp_x.wait()
    cp_l.wait()
    for p in range(NPIECE):
        lo_el = p * piece

        @functools.partial(plsc.parallel_loop, 0, piece // L, unroll=8)
        def _(i):
            x = x_v[pl.ds(lo_el + i * L, L)]
            lab = lab_v[pl.ds(lo_el + i * L, L)]
            s = one / (one + jnp.exp(-x))
            t = s * 64.0
            idx = t.astype(jnp.int32)
            frac = t - idx.astype(jnp.float32)
            flat = lane_base + idx + lab * HB
            plsc.addupdate_scatter(h2_v, [flat], one - frac)
            plsc.addupdate_scatter(h2_v, [flat + 1], frac)

    for c in range(BINS // L):
        acc = h2_v[pl.ds(c * L, L)]
        for lane in range(1, L):
            acc = acc + h2_v[pl.ds(lane * BINS + c * L, L)]
        h1_v[pl.ds(c * L, L)] = acc

    pltpu.sync_copy(h1_v, out_hbm.at[wid])


def _sc_partial_hist(logits, labels):
    n = logits.shape[0]
    mesh = plsc.VectorSubcoreMesh(core_axis_name="c", subcore_axis_name="s")
    chunk = n // NW
    f = pl.kernel(
        _sc_hist_body,
        out_type=jax.ShapeDtypeStruct((NW, BINS), jnp.float32),
        mesh=mesh,
        scratch_types=[
            pltpu.VMEM((chunk,), jnp.float32),
            pltpu.VMEM((chunk,), jnp.int32),
            pltpu.VMEM((L * BINS,), jnp.float32),
            pltpu.VMEM((BINS,), jnp.float32),
        ] + [pltpu.SemaphoreType.DMA] * (2 * NPIECE),
        compiler_params=pltpu.CompilerParams(needs_layout_passes=False),
    )
    return f(logits, labels)


def _tc_loss_body(p_ref, o_ref):
    h = jnp.sum(p_ref[...], axis=0, keepdims=True) * BIN_WIDTH
    col = lax.broadcasted_iota(jnp.int32, (1, BINS), 1)
    valid_u = col < (NUM_BINS + 1)
    valid_p = (col >= HB) & (col < HB + NUM_BINS + 1)
    hu_sum = jnp.sum(jnp.where(valid_u, h, 0.0))
    hp_sum = jnp.sum(jnp.where(valid_p, h, 0.0))
    proxy_u = jnp.where(col == 0, 1.0 - PRIOR, 0.0) + jnp.where(
        col == NUM_BINS, PRIOR, 0.0)
    proxy_p = jnp.where(col == HB + NUM_BINS, 1.0, 0.0)
    lu = jnp.sum(
        jnp.where(valid_u, jnp.abs(h / (hu_sum + 1e-8) - proxy_u), 0.0))
    lp = jnp.sum(
        jnp.where(valid_p, jnp.abs(h / (hp_sum + 1e-8) - proxy_p), 0.0))
    o_ref[0, 0] = (lp + FRAC_PRIOR * lu) / (NUM_BINS + 1.0)


def _tc_loss(partials):
    f = pl.pallas_call(
        _tc_loss_body,
        out_shape=jax.ShapeDtypeStruct((1, 1), jnp.float32),
        in_specs=[pl.BlockSpec(memory_space=pltpu.VMEM)],
        out_specs=pl.BlockSpec(memory_space=pltpu.SMEM),
    )
    return f(partials)


@jax.jit
def kernel(logits, labels):
    labels_i32 = labels.astype(jnp.int32)
    partials = _sc_partial_hist(logits, labels_i32)
    out = _tc_loss(partials)
    return out[0, 0]
